# baseline (speedup 1.0000x reference)
import jax
import jax.numpy as jnp
from jax import lax
from jax.experimental import pallas as pl
from jax.experimental.pallas import tpu as pltpu
from jax.experimental.pallas import tpu_sc as plsc

_TOPK = 5
_B = 1024
_N = 100000
_HALF = _N // 2
_NW = 32
_RPW = _B // _NW
_LEADV = 1024 // 16
_CV = _HALF // 16


def _sc_count(pred_hbm, tgt_hbm, dom_hbm, out_hbm,
              buf0, buf1, tgt_v, dom_v, red_v, out_v, sem0, sem1):
    nc = 2
    wid = lax.axis_index("s") * nc + lax.axis_index("c")
    base = wid * _RPW

    pltpu.sync_copy(tgt_hbm.at[pl.ds(base, _RPW)], tgt_v.at[pl.ds(0, _RPW)])
    pltpu.sync_copy(dom_hbm, dom_v)

    def dma0(row):
        return pltpu.make_async_copy(
            pred_hbm.at[pl.ds(row * _N, _HALF)], buf0, sem0)

    def dma1(row):
        return pltpu.make_async_copy(
            pred_hbm.at[pl.ds(row * _N + _HALF, _HALF)], buf1, sem1)

    dma0(base).start()
    dma1(base).start()

    lanes = lax.iota(jnp.int32, 16)

    def lane_sum(vec):
        red_v[pl.ds(0, 16)] = vec
        s = red_v[pl.ds(0, 16)][0]
        for k in range(1, 16):
            s = s + red_v[pl.ds(k, 16)][0]
        return s

    def row_body(r, carry):
        csum, msum = carry
        t = tgt_v[pl.ds(r, 16)][0]
        t_vec = lax.broadcast(t, (16,))
        row_next = jnp.minimum(r + 1, _RPW - 1) + base

        dma0(0).wait()
        p_t = lax.broadcast(buf0[pl.ds(t, 16)][0], (16,))

        zero16 = jnp.zeros((16,), jnp.float32)

        def lead_body(i, a):
            v = buf0[pl.ds(i * 16, 16)]
            idx = lanes + i * 16
            m = (v > p_t) | ((v == p_t) & (idx < t_vec))
            return a + jnp.where(m, 1.0, 0.0)
        acc = lax.fori_loop(0, _LEADV, lead_body, zero16, unroll=8)

        def gt0_body(i, a):
            v = buf0[pl.ds(i * 16, 16)]
            return a + jnp.where(v > p_t, 1.0, 0.0)
        acc = lax.fori_loop(_LEADV, _CV, gt0_body, acc, unroll=8)
        dma0(row_next).start()

        dma1(0).wait()

        def gt1_body(i, a):
            v = buf1[pl.ds(i * 16, 16)]
            return a + jnp.where(v > p_t, 1.0, 0.0)
        acc = lax.fori_loop(0, _CV, gt1_body, acc, unroll=8)
        dma1(row_next).start()

        dacc = zero16
        for d in range(8):
            dom_i = dom_v[pl.ds(d * 16, 16)].astype(jnp.int32)
            dacc = dacc + jnp.where(dom_i == t_vec, 1.0, 0.0)

        cnt = lane_sum(acc)
        dsum = lane_sum(dacc)
        in_dom = dsum > 0.0
        okf = jnp.where((cnt < float(_TOPK)) & in_dom, 1.0, 0.0)
        return csum + okf, msum + jnp.where(in_dom, 1.0, 0.0)

    csum, msum = lax.fori_loop(0, _RPW, row_body, (0.0, 0.0))
    dma0(0).wait()
    dma1(0).wait()

    out_v[...] = jnp.where(lanes == 0, lax.broadcast(csum, (16,)),
                           jnp.where(lanes == 1, lax.broadcast(msum, (16,)),
                                     0.0))
    pltpu.sync_copy(out_v, out_hbm.at[wid])


def kernel(prediction, target, domain):
    pred_flat = prediction.reshape(-1)
    t32 = target.astype(jnp.int32)
    dom = domain.astype(jnp.float32)
    mesh = plsc.VectorSubcoreMesh(core_axis_name="c", subcore_axis_name="s")
    partials = pl.kernel(
        _sc_count,
        mesh=mesh,
        out_type=jax.ShapeDtypeStruct((_NW, 16), jnp.float32),
        scratch_types=[
            pltpu.VMEM((_HALF,), jnp.float32),
            pltpu.VMEM((_HALF,), jnp.float32),
            pltpu.VMEM((_RPW + 16,), jnp.int32),
            pltpu.VMEM((128,), jnp.float32),
            pltpu.VMEM((32,), jnp.float32),
            pltpu.VMEM((16,), jnp.float32),
            pltpu.SemaphoreType.DMA,
            pltpu.SemaphoreType.DMA,
        ],
    )(pred_flat, t32, dom)
    csum = jnp.sum(partials[:, 0])
    msum = jnp.sum(partials[:, 1])
    return csum * (100.0 / msum)


# speedup vs baseline: 1.0382x; 1.0382x over previous
import functools

import jax
import jax.numpy as jnp
from jax import lax
from jax.experimental import pallas as pl
from jax.experimental.pallas import tpu as pltpu
from jax.experimental.pallas import tpu_sc as plsc

_TOPK = 5
_B = 1024
_N = 100000
_TMAX = 1024

_B_TC = 768
_R_TC = 32
_HALF = _N // 2
_NW = 32
_B_SC = _B - _B_TC
_RPW = _B_SC // _NW
_LEADV = _TMAX // 16
_CV = _HALF // 16


def _tc_kernel(pred_ref, tgt_ref, dom_ref, out_ref, acc_ref, *, NR, N):
    pi = pl.program_id(0)
    block = pred_ref[...]
    t = tgt_ref[...]

    lead = block[:, :_TMAX]
    lcols = jax.lax.broadcasted_iota(jnp.int32, (1, _TMAX), 1)
    onehot = lcols == t
    p_t = jnp.sum(jnp.where(onehot, lead, 0.0), axis=1, keepdims=True)

    cols = jax.lax.broadcasted_iota(jnp.int32, (1, N), 1)
    gt = (block > p_t) & (cols < N)
    cnt = jnp.sum(jnp.where(gt, 1.0, 0.0), axis=1, keepdims=True)
    eq = (lead == p_t) & (lcols < t)
    cnt = cnt + jnp.sum(jnp.where(eq, 1.0, 0.0), axis=1, keepdims=True)

    correct = (cnt < float(_TOPK)).astype(jnp.float32)
    dom_i = dom_ref[...].astype(jnp.int32)
    in_dom = jnp.sum((t == dom_i).astype(jnp.float32), axis=1, keepdims=True)
    mask = (in_dom > 0.0).astype(jnp.float32)
    msum = jnp.sum(mask, axis=0, keepdims=True)
    csum = jnp.sum(mask * correct, axis=0, keepdims=True)

    @pl.when(pi == 0)
    def _():
        acc_ref[0:1, 0:1] = csum
        acc_ref[0:1, 1:2] = msum

    @pl.when(pi > 0)
    def _():
        acc_ref[0:1, 0:1] = acc_ref[0:1, 0:1] + csum
        acc_ref[0:1, 1:2] = acc_ref[0:1, 1:2] + msum

    @pl.when(pi == NR - 1)
    def _():
        out_ref[0:1, 0:1] = acc_ref[0:1, 0:1]
        out_ref[0:1, 1:2] = acc_ref[0:1, 1:2]


def _tc_partial(pred_tc, t2, dom):
    NR = _B_TC // _R_TC
    N = pred_tc.shape[1]
    return pl.pallas_call(
        functools.partial(_tc_kernel, NR=NR, N=N),
        grid=(NR,),
        in_specs=[
            pl.BlockSpec((_R_TC, N), lambda i: (i, 0)),
            pl.BlockSpec((_R_TC, 1), lambda i: (i, 0)),
            pl.BlockSpec((1, dom.shape[1]), lambda i: (0, 0)),
        ],
        out_specs=pl.BlockSpec((1, 2), lambda i: (0, 0)),
        out_shape=jax.ShapeDtypeStruct((1, 2), jnp.float32),
        scratch_shapes=[pltpu.VMEM((8, 128), jnp.float32)],
    )(pred_tc, t2, dom)


def _sc_count(pred_hbm, tgt_hbm, dom_hbm, out_hbm,
              buf0, buf1, tgt_v, dom_v, red_v, out_v, sem0, sem1):
    nc = 2
    wid = lax.axis_index("s") * nc + lax.axis_index("c")
    base = _B_TC + wid * _RPW

    pltpu.sync_copy(tgt_hbm.at[pl.ds(base, _RPW)], tgt_v.at[pl.ds(0, _RPW)])
    pltpu.sync_copy(dom_hbm, dom_v)

    def dma0(row):
        return pltpu.make_async_copy(
            pred_hbm.at[pl.ds(row * _N, _HALF)], buf0, sem0)

    def dma1(row):
        return pltpu.make_async_copy(
            pred_hbm.at[pl.ds(row * _N + _HALF, _HALF)], buf1, sem1)

    dma0(base).start()
    dma1(base).start()

    lanes = lax.iota(jnp.int32, 16)

    def lane_sum(vec):
        red_v[pl.ds(0, 16)] = vec
        s = red_v[pl.ds(0, 16)][0]
        for k in range(1, 16):
            s = s + red_v[pl.ds(k, 16)][0]
        return s

    def row_body(r, carry):
        csum, msum = carry
        t = tgt_v[pl.ds(r, 16)][0]
        t_vec = lax.broadcast(t, (16,))
        row_next = jnp.minimum(r + 1, _RPW - 1) + base

        dma0(0).wait()
        p_t = lax.broadcast(buf0[pl.ds(t, 16)][0], (16,))

        zero16 = jnp.zeros((16,), jnp.float32)

        def lead_body(i, a):
            v = buf0[pl.ds(i * 16, 16)]
            idx = lanes + i * 16
            m = (v == p_t) & (idx < t_vec)
            return a + jnp.where(m, 1.0, 0.0)
        acc = lax.fori_loop(0, _LEADV, lead_body, zero16, unroll=8)

        def make_gt_body(buf):
            def gt_body(i, accs):
                b = i * 80
                return tuple(
                    a + jnp.where(buf[pl.ds(b + 16 * k, 16)] > p_t, 1.0, 0.0)
                    for k, a in enumerate(accs))
            return gt_body

        accs = (acc, zero16, zero16, zero16, zero16)
        accs = lax.fori_loop(0, _CV // 5, make_gt_body(buf0), accs, unroll=2)
        dma0(row_next).start()

        dma1(0).wait()
        accs = lax.fori_loop(0, _CV // 5, make_gt_body(buf1), accs, unroll=2)
        dma1(row_next).start()

        acc = sum(accs[1:], accs[0])

        dacc = zero16
        for d in range(8):
            dom_i = dom_v[pl.ds(d * 16, 16)].astype(jnp.int32)
            dacc = dacc + jnp.where(dom_i == t_vec, 1.0, 0.0)

        cnt = lane_sum(acc)
        dsum = lane_sum(dacc)
        in_dom = dsum > 0.0
        okf = jnp.where((cnt < float(_TOPK)) & in_dom, 1.0, 0.0)
        return csum + okf, msum + jnp.where(in_dom, 1.0, 0.0)

    csum, msum = lax.fori_loop(0, _RPW, row_body, (0.0, 0.0))
    dma0(0).wait()
    dma1(0).wait()

    out_v[...] = jnp.where(lanes == 0, lax.broadcast(csum, (16,)),
                           jnp.where(lanes == 1, lax.broadcast(msum, (16,)),
                                     0.0))
    pltpu.sync_copy(out_v, out_hbm.at[wid])


def _sc_partial(pred_flat, t32, dom1d):
    mesh = plsc.VectorSubcoreMesh(core_axis_name="c", subcore_axis_name="s")
    return pl.kernel(
        _sc_count,
        mesh=mesh,
        out_type=jax.ShapeDtypeStruct((_NW, 16), jnp.float32),
        scratch_types=[
            pltpu.VMEM((_HALF,), jnp.float32),
            pltpu.VMEM((_HALF,), jnp.float32),
            pltpu.VMEM((_RPW + 16,), jnp.int32),
            pltpu.VMEM((128,), jnp.float32),
            pltpu.VMEM((32,), jnp.float32),
            pltpu.VMEM((16,), jnp.float32),
            pltpu.SemaphoreType.DMA,
            pltpu.SemaphoreType.DMA,
        ],
    )(pred_flat, t32, dom1d)


def kernel(prediction, target, domain):
    B, N = prediction.shape
    t32 = target.astype(jnp.int32)
    t2 = t32.reshape(B, 1)
    dom2 = domain.reshape(1, -1).astype(jnp.float32)

    tc = _tc_partial(prediction, t2, dom2)
    sc = _sc_partial(prediction.reshape(-1), t32,
                     domain.astype(jnp.float32))

    csum = tc[0, 0] + jnp.sum(sc[:, 0])
    msum = tc[0, 1] + jnp.sum(sc[:, 1])
    return csum * (100.0 / msum)


# speedup vs baseline: 2.2291x; 2.1470x over previous
import functools

import jax
import jax.numpy as jnp
from jax import lax
from jax.experimental import pallas as pl
from jax.experimental.pallas import tpu as pltpu
from jax.experimental.pallas import tpu_sc as plsc

_TOPK = 5
_B = 1024
_N = 100000
_TMAX = 1024

_B_TC = 768
_R_TC = 32
_HALF = _N // 2
_NW = 32
_B_SC = _B - _B_TC
_RPW = _B_SC // _NW
_LEADV = _TMAX // 16
_CV = _HALF // 16


def _tc_kernel(pred_ref, tgt_ref, dom_ref, out_ref, acc_ref, *, NR, N):
    pi = pl.program_id(0)
    block = pred_ref[...]
    t = tgt_ref[...]

    lead = block[:, :_TMAX]
    lcols = jax.lax.broadcasted_iota(jnp.int32, (1, _TMAX), 1)
    onehot = lcols == t
    p_t = jnp.sum(jnp.where(onehot, lead, 0.0), axis=1, keepdims=True)

    cols = jax.lax.broadcasted_iota(jnp.int32, (1, N), 1)
    gt = (block > p_t) & (cols < N)
    cnt = jnp.sum(jnp.where(gt, 1.0, 0.0), axis=1, keepdims=True)
    eq = (lead == p_t) & (lcols < t)
    cnt = cnt + jnp.sum(jnp.where(eq, 1.0, 0.0), axis=1, keepdims=True)

    correct = (cnt < float(_TOPK)).astype(jnp.float32)
    dom_i = dom_ref[...].astype(jnp.int32)
    in_dom = jnp.sum((t == dom_i).astype(jnp.float32), axis=1, keepdims=True)
    mask = (in_dom > 0.0).astype(jnp.float32)
    msum = jnp.sum(mask, axis=0, keepdims=True)
    csum = jnp.sum(mask * correct, axis=0, keepdims=True)

    @pl.when(pi == 0)
    def _():
        acc_ref[0:1, 0:1] = csum
        acc_ref[0:1, 1:2] = msum

    @pl.when(pi > 0)
    def _():
        acc_ref[0:1, 0:1] = acc_ref[0:1, 0:1] + csum
        acc_ref[0:1, 1:2] = acc_ref[0:1, 1:2] + msum

    @pl.when(pi == NR - 1)
    def _():
        out_ref[0:1, 0:1] = acc_ref[0:1, 0:1]
        out_ref[0:1, 1:2] = acc_ref[0:1, 1:2]


def _tc_partial(pred_tc, t2, dom):
    NR = _B_TC // _R_TC
    N = pred_tc.shape[1]
    return pl.pallas_call(
        functools.partial(_tc_kernel, NR=NR, N=N),
        grid=(NR,),
        in_specs=[
            pl.BlockSpec((_R_TC, N), lambda i: (i, 0)),
            pl.BlockSpec((_R_TC, 1), lambda i: (i, 0)),
            pl.BlockSpec((1, dom.shape[1]), lambda i: (0, 0)),
        ],
        out_specs=pl.BlockSpec((1, 2), lambda i: (0, 0)),
        out_shape=jax.ShapeDtypeStruct((1, 2), jnp.float32),
        scratch_shapes=[pltpu.VMEM((8, 128), jnp.float32)],
    )(pred_tc, t2, dom)


_CW = 6144
_NCH = 16
_TAILO = _NCH * _CW
_TAILW = 1664
_EDGEO = _TAILO + _TAILW
_EDGEW = _N - _EDGEO


def _sc_count(pred_hbm, tgt_hbm, dom_hbm, out_hbm,
              buf0, buf1, tailb, edgeb, tgt_v, dom_v, red_v, out_v,
              sem0, sem1, semt, seme):
    nc = 2
    wid = lax.axis_index("s") * nc + lax.axis_index("c")
    rbase = _B_TC + wid * _RPW

    pltpu.sync_copy(tgt_hbm.at[pl.ds(rbase, _RPW)], tgt_v.at[pl.ds(0, _RPW)])
    pltpu.sync_copy(dom_hbm, dom_v)

    bufs = (buf0, buf1)
    sems = (sem0, sem1)

    def dma(c):
        return pltpu.make_async_copy(
            pred_hbm.at[pl.ds(rbase, _RPW), pl.ds(c * _CW, _CW)],
            bufs[c % 2], sems[c % 2])

    dma_tail = pltpu.make_async_copy(
        pred_hbm.at[pl.ds(rbase, _RPW), pl.ds(_TAILO, _TAILW)], tailb, semt)
    dma_edge = pltpu.make_async_copy(
        pred_hbm.at[pl.ds(rbase, _RPW), pl.ds(_EDGEO, _EDGEW)], edgeb, seme)

    dma(0).start()
    dma(1).start()
    dma_tail.start()
    dma_edge.start()

    lanes = lax.iota(jnp.int32, 16)
    zero16 = jnp.zeros((16,), jnp.float32)

    def lane_sum(vec):
        red_v[pl.ds(0, 16)] = vec
        s = red_v[pl.ds(0, 16)][0]
        for k in range(1, 16):
            s = s + red_v[pl.ds(k, 16)][0]
        return s

    accs = [zero16] * _RPW
    p_ts = [None] * _RPW
    t_scals = [tgt_v[pl.ds(r, 16)][0] for r in range(_RPW)]

    for c in range(_NCH):
        dma(c).wait()
        buf = bufs[c % 2]
        if c == 0:
            for r in range(_RPW):
                t_vec = lax.broadcast(t_scals[r], (16,))

                def pt_body(i, a, r=r, t_vec=t_vec):
                    v = buf[r, pl.ds(i * 16, 16)]
                    idx = lanes + i * 16
                    return a + jnp.where(idx == t_vec, v, 0.0)
                pacc = lax.fori_loop(0, _LEADV, pt_body, zero16, unroll=8)
                p_ts[r] = lax.broadcast(lane_sum(pacc), (16,))

                def eq_body(i, a, r=r, t_vec=t_vec):
                    v = buf[r, pl.ds(i * 16, 16)]
                    idx = lanes + i * 16
                    m = (v == p_ts[r]) & (idx < t_vec)
                    return a + jnp.where(m, 1.0, 0.0)
                accs[r] = lax.fori_loop(0, _LEADV, eq_body, accs[r], unroll=8)

        def gt_body(i, acc_t, buf=buf):
            return tuple(
                a + jnp.where(buf[r, pl.ds(i * 16, 16)] > p_ts[r], 1.0, 0.0)
                for r, a in enumerate(acc_t))
        accs = list(lax.fori_loop(0, _CW // 16, gt_body, tuple(accs),
                                  unroll=2))
        if c + 2 < _NCH:
            dma(c + 2).start()

    dma_tail.wait()

    def tail_body(i, acc_t):
        return tuple(
            a + jnp.where(tailb[r, pl.ds(i * 16, 16)] > p_ts[r], 1.0, 0.0)
            for r, a in enumerate(acc_t))
    accs = list(lax.fori_loop(0, _TAILW // 16, tail_body, tuple(accs),
                              unroll=2))

    dma_edge.wait()
    for r in range(_RPW):
        for i in range(_EDGEW // 16):
            v = edgeb[r, pl.ds(i * 16, 16)]
            accs[r] = accs[r] + jnp.where(v > p_ts[r], 1.0, 0.0)

    csum = 0.0
    msum = 0.0
    for r in range(_RPW):
        t_vec = lax.broadcast(t_scals[r], (16,))
        dacc = zero16
        for d in range(8):
            dom_i = dom_v[pl.ds(d * 16, 16)].astype(jnp.int32)
            dacc = dacc + jnp.where(dom_i == t_vec, 1.0, 0.0)
        cnt = lane_sum(accs[r])
        dsum = lane_sum(dacc)
        in_dom = dsum > 0.0
        okf = jnp.where((cnt < float(_TOPK)) & in_dom, 1.0, 0.0)
        csum = csum + okf
        msum = msum + jnp.where(in_dom, 1.0, 0.0)

    out_v[...] = jnp.where(lanes == 0, lax.broadcast(csum, (16,)),
                           jnp.where(lanes == 1, lax.broadcast(msum, (16,)),
                                     0.0))
    pltpu.sync_copy(out_v, out_hbm.at[wid])


def _sc_partial(pred, t32, dom1d):
    mesh = plsc.VectorSubcoreMesh(core_axis_name="c", subcore_axis_name="s")
    return pl.kernel(
        _sc_count,
        mesh=mesh,
        out_type=jax.ShapeDtypeStruct((_NW, 16), jnp.float32),
        scratch_types=[
            pltpu.VMEM((_RPW, _CW), jnp.float32),
            pltpu.VMEM((_RPW, _CW), jnp.float32),
            pltpu.VMEM((_RPW, _TAILW), jnp.float32),
            pltpu.VMEM((_RPW, _EDGEW), jnp.float32),
            pltpu.VMEM((_RPW + 16,), jnp.int32),
            pltpu.VMEM((128,), jnp.float32),
            pltpu.VMEM((32,), jnp.float32),
            pltpu.VMEM((16,), jnp.float32),
            pltpu.SemaphoreType.DMA,
            pltpu.SemaphoreType.DMA,
            pltpu.SemaphoreType.DMA,
            pltpu.SemaphoreType.DMA,
        ],
    )(pred, t32, dom1d)


def kernel(prediction, target, domain):
    B, N = prediction.shape
    t32 = target.astype(jnp.int32)
    t2 = t32.reshape(B, 1)
    dom2 = domain.reshape(1, -1).astype(jnp.float32)

    tc = _tc_partial(prediction, t2, dom2)
    sc = _sc_partial(prediction, t32,
                     domain.astype(jnp.float32))

    csum = tc[0, 0] + jnp.sum(sc[:, 0])
    msum = tc[0, 1] + jnp.sum(sc[:, 1])
    return csum * (100.0 / msum)
